# baseline (speedup 1.0000x reference)
import jax
import jax.numpy as jnp
from jax.experimental import pallas as pl
from jax.experimental.pallas import tpu as pltpu

N = 1048576
IN_CH = 9
OUT_CH = 64
BN_EPS = 1e-3
G = N // 8
ROWSG = 2048


@jax.jit
def kernel(features, W, gamma, beta):
    x3 = features.reshape(G, 8, IN_CH)
    wb = W.astype(jnp.bfloat16)
    g2 = gamma.reshape(1, OUT_CH)
    b2 = beta.reshape(1, OUT_CH)

    def body(x_ref, w_ref, g_ref, b_ref, o_ref, acc):
        p = pl.program_id(0)
        t = pl.program_id(1)
        xb = x_ref[...].reshape(ROWSG * 8, IN_CH).astype(jnp.bfloat16)
        h = jnp.dot(xb, w_ref[...], preferred_element_type=jnp.float32)

        @pl.when(p == 0)
        def _stats():
            s = jnp.sum(h, axis=0, keepdims=True)
            q = jnp.sum(h * h, axis=0, keepdims=True)
            blk = jnp.concatenate([s, q], axis=0)

            @pl.when(t == 0)
            def _init():
                acc[...] = blk

            @pl.when(t > 0)
            def _acc():
                acc[...] = acc[...] + blk

        @pl.when(p == 1)
        def _apply():
            s = acc[0:1, :]
            q = acc[1:2, :]
            mean = s * (1.0 / N)
            var = q * (1.0 / N) - mean * mean
            inv = jax.lax.rsqrt(var + BN_EPS)
            scale = g_ref[...] * inv
            bias = b_ref[...] - mean * scale
            o = jnp.maximum(h * scale + bias, 0.0)
            o_ref[...] = o.reshape(ROWSG, 8, OUT_CH)

    out3 = pl.pallas_call(
        body,
        grid=(2, G // ROWSG),
        in_specs=[
            pl.BlockSpec((ROWSG, 8, IN_CH), lambda p, t: (t, 0, 0)),
            pl.BlockSpec((IN_CH, OUT_CH), lambda p, t: (0, 0)),
            pl.BlockSpec((1, OUT_CH), lambda p, t: (0, 0)),
            pl.BlockSpec((1, OUT_CH), lambda p, t: (0, 0)),
        ],
        out_specs=pl.BlockSpec((ROWSG, 8, OUT_CH), lambda p, t: (t * p, 0, 0)),
        out_shape=jax.ShapeDtypeStruct((G, 8, OUT_CH), jnp.float32),
        scratch_shapes=[pltpu.VMEM((2, OUT_CH), jnp.float32)],
    )(x3, wb, g2, b2)
    return out3.reshape(N, OUT_CH)
